# baseline (speedup 1.0000x reference)
import functools

import jax
import jax.numpy as jnp
from jax import lax
from jax.experimental import pallas as pl
from jax.experimental.pallas import tpu as pltpu

N_CLASSES = 10000
D_IN = 512
N_FEAT = 128
BATCH = 1024

BC = 2000
NB = N_CLASSES // BC
BR = 128
NR = BATCH // BR


def _argmax_body(x_ref, w_ref, proto_ref, cp_ref, predsT_ref, rmax_ref,
                 rarg_ref):
    j = pl.program_id(0)

    @pl.when(j == 0)
    def _init():
        featT = lax.dot_general(
            w_ref[...], x_ref[...],
            dimension_numbers=(((0,), (1,)), ((), ())),
            preferred_element_type=jnp.float32)
        norm = jnp.maximum(
            jnp.sqrt(jnp.sum(featT * featT, axis=0, keepdims=True)), 1e-12)
        predsT_ref[...] = featT / norm
        rmax_ref[...] = jnp.full((1, BATCH), -jnp.inf, jnp.float32)
        rarg_ref[...] = jnp.zeros((1, BATCH), jnp.int32)

    s = lax.dot_general(
        proto_ref[...], predsT_ref[...],
        dimension_numbers=(((1,), (0,)), ((), ())),
        preferred_element_type=jnp.float32)
    m = jnp.max(s, axis=0, keepdims=True)
    row = lax.broadcasted_iota(jnp.int32, (BC, BATCH), 0)
    arg = jnp.min(jnp.where(s == m, row, BC), axis=0,
                  keepdims=True) + j * BC
    better = m > rmax_ref[...]
    rarg_ref[...] = jnp.where(better, arg, rarg_ref[...])
    rmax_ref[...] = jnp.where(better, m, rmax_ref[...])

    @pl.when(j == NB - 1)
    def _done():
        cp_ref[...] = rarg_ref[...]


def _classpred(x, W, prototypes):
    return pl.pallas_call(
        _argmax_body,
        grid=(NB,),
        in_specs=[
            pl.BlockSpec((BATCH, D_IN), lambda j: (0, 0)),
            pl.BlockSpec((D_IN, N_FEAT), lambda j: (0, 0)),
            pl.BlockSpec((BC, N_FEAT), lambda j: (j, 0)),
        ],
        out_specs=pl.BlockSpec((1, BATCH), lambda j: (0, 0)),
        out_shape=jax.ShapeDtypeStruct((1, BATCH), jnp.int32),
        scratch_shapes=[
            pltpu.VMEM((N_FEAT, BATCH), jnp.float32),
            pltpu.VMEM((1, BATCH), jnp.float32),
            pltpu.VMEM((1, BATCH), jnp.int32),
        ],
    )(x, W, prototypes)


def _onehot_body(cp_ref, out_ref):
    col = lax.broadcasted_iota(jnp.int32, (BR, N_CLASSES), 1)
    out_ref[...] = jnp.where(col == cp_ref[...], 1.0, 0.0).astype(jnp.float32)


def _onehot(cp):
    return pl.pallas_call(
        _onehot_body,
        grid=(NR,),
        in_specs=[pl.BlockSpec((BR, 1), lambda i: (i, 0))],
        out_specs=pl.BlockSpec((BR, N_CLASSES), lambda i: (i, 0)),
        out_shape=jax.ShapeDtypeStruct((BATCH, N_CLASSES), jnp.float32),
    )(cp)


@jax.jit
def _run(x, W, prototypes):
    cp = _classpred(x, W, prototypes)
    return _onehot(cp.reshape(BATCH, 1))


def kernel(x, t, W, prototypes):
    return _run(x, W, prototypes)
